# baseline (speedup 1.0000x reference)
import jax
import jax.numpy as jnp
from jax import lax
from jax.experimental import pallas as pl
from jax.experimental.pallas import tpu as pltpu
from jax.experimental.pallas import tpu_sc as plsc

BSZ = 64
BEAMS = 4
VOCAB = 100000
K_OUT = 8

LANES = 16
CHUNK = 20000
CHUNKS_PER_ROW = BEAMS * VOCAB // CHUNK
CHUNKS_PER_BEAM = VOCAB // CHUNK
GROUP_VREGS = 25
GROUP = GROUP_VREGS * LANES
NGROUPS = CHUNK // GROUP

NWORK = 32
ROWS_PER_W = BSZ // NWORK

NEG_INF = float("-inf")


def _splat0(x):
    return jnp.broadcast_to(x[0], (LANES,))


def _any(mask):
    return plsc.all_reduce_population_count(mask)[0] > 0


def _merge_topk(T, TI, v, vi):
    vd, vdi = plsc.sort_key_val(v, vi, descending=True)
    keep = (T > vd) | ((T == vd) & (TI < vdi))
    newT = jnp.maximum(T, vd)
    newTI = jnp.where(keep, TI, vdi)
    sT, sTI = plsc.sort_key_val(newT, newTI, descending=False)
    return sT, sTI


def _tile_body(lprobs_hbm, bias_hbm, vals_hbm, toks_hbm, beams_hbm,
               buf0, buf1, bias_v, out_v, out_ti, out_bi, sem0, sem1):
    cid = lax.axis_index("c")
    sid = lax.axis_index("s")
    wid = sid * 2 + cid

    iota = lax.iota(jnp.int32, LANES)

    def process_chunk(buf, t, T, TI):
        biasv = bias_v[t // CHUNKS_PER_BEAM]
        off = t * CHUNK
        thr_vec = _splat0(T) - biasv

        def group_body(g, carry):
            T, TI, thr_vec = carry
            base = g * GROUP
            gmax = buf[pl.ds(base, LANES)]
            for j in range(1, GROUP_VREGS):
                gmax = jnp.maximum(gmax, buf[pl.ds(base + j * LANES, LANES)])

            def insert(carry):
                T, TI, thr_vec = carry

                def ins_j(j, c3):
                    T, TI, thr_vec = c3
                    p = base + j * LANES
                    v = buf[pl.ds(p, LANES)]

                    def do_merge(c4):
                        T, TI, _ = c4
                        vb = v + biasv
                        vi = iota + (off + p)
                        T2, TI2 = _merge_topk(T, TI, vb, vi)
                        return T2, TI2, _splat0(T2) - biasv

                    return lax.cond(jnp.any(v > thr_vec), do_merge,
                                    lambda c4: c4, (T, TI, thr_vec))

                return lax.fori_loop(0, GROUP_VREGS, ins_j, (T, TI, thr_vec))

            return lax.cond(jnp.any(gmax > thr_vec), insert,
                            lambda c: c, (T, TI, thr_vec))

        T, TI, _ = lax.fori_loop(0, NGROUPS, group_body, (T, TI, thr_vec))
        return T, TI

    for r in range(ROWS_PER_W):
        row = wid * ROWS_PER_W + r
        pltpu.sync_copy(bias_hbm.at[row], bias_v)

        T = jnp.full((LANES,), NEG_INF, jnp.float32)
        TI = jnp.zeros((LANES,), jnp.int32)

        src = lambda t: lprobs_hbm.at[row, pl.ds(t * CHUNK, CHUNK)]
        pltpu.async_copy(src(0), buf0, sem0)
        pltpu.async_copy(src(1), buf1, sem1)

        def pair_body(i, carry, row=row):
            T, TI = carry
            t0 = 2 * i
            pltpu.make_async_copy(src(0), buf0, sem0).wait()
            T, TI = process_chunk(buf0, t0, T, TI)
            pltpu.async_copy(
                lprobs_hbm.at[row, pl.ds((t0 + 2) * CHUNK, CHUNK)], buf0, sem0)
            t1 = 2 * i + 1
            pltpu.make_async_copy(src(1), buf1, sem1).wait()
            T, TI = process_chunk(buf1, t1, T, TI)
            pltpu.async_copy(
                lprobs_hbm.at[row, pl.ds((t1 + 2) * CHUNK, CHUNK)], buf1, sem1)
            return T, TI

        T, TI = lax.fori_loop(0, (CHUNKS_PER_ROW - 2) // 2, pair_body, (T, TI))
        pltpu.make_async_copy(src(0), buf0, sem0).wait()
        T, TI = process_chunk(buf0, CHUNKS_PER_ROW - 2, T, TI)
        pltpu.make_async_copy(src(1), buf1, sem1).wait()
        T, TI = process_chunk(buf1, CHUNKS_PER_ROW - 1, T, TI)

        Td = lax.rev(T, dimensions=(0,))
        TId = lax.rev(TI, dimensions=(0,))
        out_v[...] = Td
        out_ti[...] = TId % VOCAB
        out_bi[...] = TId // VOCAB
        pltpu.sync_copy(out_v, vals_hbm.at[row])
        pltpu.sync_copy(out_ti, toks_hbm.at[row])
        pltpu.sync_copy(out_bi, beams_hbm.at[row])


@jax.jit
def _topk_sc(lprobs_flat, bias_splat):
    kern = pl.kernel(
        _tile_body,
        out_type=(
            jax.ShapeDtypeStruct((BSZ, LANES), jnp.float32),
            jax.ShapeDtypeStruct((BSZ, LANES), jnp.int32),
            jax.ShapeDtypeStruct((BSZ, LANES), jnp.int32),
        ),
        mesh=plsc.VectorSubcoreMesh(core_axis_name="c", subcore_axis_name="s"),
        scratch_types=[
            pltpu.VMEM((CHUNK,), jnp.float32),
            pltpu.VMEM((CHUNK,), jnp.float32),
            pltpu.VMEM((BEAMS, LANES), jnp.float32),
            pltpu.VMEM((LANES,), jnp.float32),
            pltpu.VMEM((LANES,), jnp.int32),
            pltpu.VMEM((LANES,), jnp.int32),
            pltpu.SemaphoreType.DMA,
            pltpu.SemaphoreType.DMA,
        ],
        compiler_params=pltpu.CompilerParams(use_tc_tiling_on_sc=False,
                                             needs_layout_passes=False),
    )
    return kern(lprobs_flat, bias_splat)


def kernel(step_in_seq, lprobs, scores):
    bsz, beam_size, vocab = lprobs.shape
    bias = lax.dynamic_index_in_dim(scores, step_in_seq - 1, axis=2,
                                    keepdims=False)
    bias_splat = jnp.broadcast_to(bias[:, :, None], (bsz, beam_size, LANES))
    lprobs_flat = lprobs.reshape(bsz, beam_size * vocab)
    vals, toks, beams = _topk_sc(lprobs_flat, bias_splat)
    return (vals[:, :K_OUT], toks[:, :K_OUT], beams[:, :K_OUT])
